# baseline (speedup 1.0000x reference)
import functools

import numpy as np
import jax
import jax.numpy as jnp
from jax import lax
from jax.experimental import pallas as pl
from jax.experimental.pallas import tpu as pltpu
from jax.experimental.pallas import tpu_sc as plsc

_EMB = 128
_B, _S = 1024, 200
_NCHUNK = 2
_CS = _S // _NCHUNK
_NC, _NS = 2, 16
_NW = _NC * _NS
_ROWS_PER_W = _B // _NW
_GROUP = 4


@functools.cache
def _enc_table():
    pos = np.arange(1, _S + 1, dtype=np.float64)[:, None]
    j = np.arange(_EMB, dtype=np.float64)[None, :]
    pe = pos / np.power(10000.0, 2.0 * (np.floor(j / 2.0)) / _EMB)
    pe[1:, 0::2] = np.sin(pe[1:, 0::2])
    pe[1:, 1::2] = np.cos(pe[1:, 1::2])
    return jnp.asarray(pe.astype(np.float32)).reshape(_NCHUNK, _CS, _EMB)


def _sc_scatter(enc, sol, best):
    mesh = plsc.VectorSubcoreMesh(core_axis_name="c", subcore_axis_name="s")

    @functools.partial(
        pl.kernel,
        mesh=mesh,
        out_type=(
            jax.ShapeDtypeStruct((_B, _S, _EMB), jnp.float32),
            jax.ShapeDtypeStruct((_B, _S, _EMB), jnp.float32),
        ),
        scratch_types=[
            pltpu.VMEM((_NCHUNK, _CS, _EMB), jnp.float32),
            pltpu.VMEM((_ROWS_PER_W, _NCHUNK, _CS), jnp.int32),
            pltpu.VMEM((_ROWS_PER_W, _NCHUNK, _CS), jnp.int32),
            pltpu.SemaphoreType.DMA,
        ],
    )
    def k(enc_hbm, sol_hbm, best_hbm, out_hbm, bout_hbm, enc_v, sidx_v, bidx_v, sem):
        wid = lax.axis_index("s") * _NC + lax.axis_index("c")
        base = wid * _ROWS_PER_W
        pltpu.sync_copy(enc_hbm, enc_v)
        pltpu.sync_copy(sol_hbm.at[pl.ds(base, _ROWS_PER_W)], sidx_v)
        pltpu.sync_copy(best_hbm.at[pl.ds(base, _ROWS_PER_W)], bidx_v)

        def group(g, carry):
            r0 = g * _GROUP
            cps = []
            for dr in range(_GROUP):
                r = r0 + dr
                for c in range(_NCHUNK):
                    cps.append(pltpu.async_copy(
                        enc_v.at[c], out_hbm.at[base + r].at[sidx_v.at[r, c]], sem))
                    cps.append(pltpu.async_copy(
                        enc_v.at[c], bout_hbm.at[base + r].at[bidx_v.at[r, c]], sem))
            for cp in cps:
                cp.wait()
            return carry

        lax.fori_loop(0, _ROWS_PER_W // _GROUP, group, 0)

    return k(enc, sol, best)


def kernel(x, solutions, best_solutions):
    del x
    sol = solutions.astype(jnp.int32).reshape(_B, _NCHUNK, _CS)
    best = best_solutions.astype(jnp.int32).reshape(_B, _NCHUNK, _CS)
    return _sc_scatter(_enc_table(), sol, best)


# speedup vs baseline: 1.0158x; 1.0158x over previous
import functools

import numpy as np
import jax
import jax.numpy as jnp
from jax import lax
from jax.experimental import pallas as pl
from jax.experimental.pallas import tpu as pltpu
from jax.experimental.pallas import tpu_sc as plsc

_EMB = 128
_B, _S = 1024, 200
_NCHUNK = 2
_CS = _S // _NCHUNK
_NC, _NS = 2, 16
_NW = _NC * _NS
_ROWS_PER_W = _B // _NW
_GROUP = 4


@functools.cache
def _enc_table():
    pos = np.arange(1, _S + 1, dtype=np.float64)[:, None]
    j = np.arange(_EMB, dtype=np.float64)[None, :]
    pe = pos / np.power(10000.0, 2.0 * (np.floor(j / 2.0)) / _EMB)
    pe[1:, 0::2] = np.sin(pe[1:, 0::2])
    pe[1:, 1::2] = np.cos(pe[1:, 1::2])
    return jnp.asarray(pe.astype(np.float32)).reshape(_NCHUNK, _CS, _EMB)


def _sc_scatter(enc, sol, best):
    mesh = plsc.VectorSubcoreMesh(core_axis_name="c", subcore_axis_name="s")

    @functools.partial(
        pl.kernel,
        mesh=mesh,
        out_type=(
            jax.ShapeDtypeStruct((_B, _S, _EMB), jnp.float32),
            jax.ShapeDtypeStruct((_B, _S, _EMB), jnp.float32),
        ),
        scratch_types=[
            pltpu.VMEM((_NCHUNK, _CS, _EMB), jnp.float32),
            pltpu.VMEM((_ROWS_PER_W, _NCHUNK, _CS), jnp.int32),
            pltpu.VMEM((_ROWS_PER_W, _NCHUNK, _CS), jnp.int32),
            pltpu.SemaphoreType.DMA,
        ],
    )
    def k(enc_hbm, sol_hbm, best_hbm, out_hbm, bout_hbm, enc_v, sidx_v, bidx_v, sem):
        wid = lax.axis_index("s") * _NC + lax.axis_index("c")
        base = wid * _ROWS_PER_W
        lds = [
            pltpu.async_copy(enc_hbm, enc_v, sem),
            pltpu.async_copy(sol_hbm.at[pl.ds(base, _ROWS_PER_W)], sidx_v, sem),
            pltpu.async_copy(best_hbm.at[pl.ds(base, _ROWS_PER_W)], bidx_v, sem),
        ]
        for ld in lds:
            ld.wait()

        def group(g, carry):
            r0 = g * _GROUP
            cps = []
            for dr in range(_GROUP):
                r = r0 + dr
                for c in range(_NCHUNK):
                    cps.append(pltpu.async_copy(
                        enc_v.at[c], out_hbm.at[base + r].at[sidx_v.at[r, c]], sem))
                    cps.append(pltpu.async_copy(
                        enc_v.at[c], bout_hbm.at[base + r].at[bidx_v.at[r, c]], sem))
            for cp in cps:
                cp.wait()
            return carry

        lax.fori_loop(0, _ROWS_PER_W // _GROUP, group, 0)

    return k(enc, sol, best)


def kernel(x, solutions, best_solutions):
    del x
    sol = solutions.astype(jnp.int32).reshape(_B, _NCHUNK, _CS)
    best = best_solutions.astype(jnp.int32).reshape(_B, _NCHUNK, _CS)
    return _sc_scatter(_enc_table(), sol, best)
